# baseline (speedup 1.0000x reference)
import functools

import jax
import jax.numpy as jnp
from jax import lax
from jax.experimental import pallas as pl
from jax.experimental.pallas import tpu as pltpu
from jax.experimental.pallas import tpu_sc as plsc

_L = 16


def _lane_allmax(v, scratch):
    iota = jnp.arange(_L, dtype=jnp.int32)
    for k in (8, 4, 2, 1):
        scratch[...] = v
        g = plsc.load_gather(scratch, [jnp.bitwise_xor(iota, k)])
        v = jnp.maximum(v, g)
    return v


def _sc_prelude_body(B, H, W, half, mask_hbm, table_hbm, out_hbm,
                     mask_v, rs_v, perm_v, idx_v, rows_v, sem):
    nc = 2
    wid = lax.axis_index("s") * nc + lax.axis_index("c")
    nchunk = W // _L
    iota = jnp.arange(_L, dtype=jnp.int32)

    @pl.when(wid < B)
    def _():
        b = wid
        pltpu.sync_copy(mask_hbm.at[b], mask_v)

        def h_body(h, cacc):
            chunks = [mask_v[h, pl.ds(_L * j, _L)] for j in range(nchunk)]
            rtot = chunks[0]
            for ch in chunks[1:]:
                rtot = rtot + ch
            rs_v[h, :] = rtot
            return tuple(c + ch for c, ch in zip(cacc, chunks))

        zero = jnp.zeros((_L,), jnp.float32)
        cacc = lax.fori_loop(0, H, h_body, (zero,) * nchunk)

        cmax = cacc[0]
        for c in cacc[1:]:
            cmax = jnp.maximum(cmax, c)
        s1v = _lane_allmax(cmax, perm_v)

        rmax = zero
        for g in range(H // _L):
            rows = g * _L + iota
            tot = plsc.load_gather(rs_v, [rows, jnp.full((_L,), 0, jnp.int32)])
            for j in range(1, _L):
                tot = tot + plsc.load_gather(
                    rs_v, [rows, jnp.full((_L,), j, jnp.int32)])
            rmax = jnp.maximum(rmax, tot)
        s2v = _lane_allmax(rmax, perm_v)

        s1i = jnp.clip(s1v.astype(jnp.int32), 0, H - 1)
        s2i = jnp.clip(s2v.astype(jnp.int32), 0, W - 1)
        idx_v[...] = s1i * W + s2i
        pltpu.async_copy(table_hbm.at[idx_v], rows_v, sem).wait()
        pltpu.sync_copy(rows_v.at[0], out_hbm.at[b, 0])


@functools.lru_cache(maxsize=None)
def _make_sc_prelude(B, H, W, half):
    mesh = plsc.VectorSubcoreMesh(core_axis_name="c", subcore_axis_name="s")
    return functools.partial(
        pl.kernel,
        mesh=mesh,
        compiler_params=pltpu.CompilerParams(needs_layout_passes=False),
        out_type=jax.ShapeDtypeStruct((B, 1, half), jnp.float32),
        scratch_types=[
            pltpu.VMEM((H, W), jnp.float32),
            pltpu.VMEM((H, _L), jnp.float32),
            pltpu.VMEM((_L,), jnp.float32),
            pltpu.VMEM((_L,), jnp.int32),
            pltpu.VMEM((_L, half), jnp.float32),
            pltpu.SemaphoreType.DMA,
        ],
    )(functools.partial(_sc_prelude_body, B, H, W, half))


def _pos_body(mask_ref, table_ref, out_ref):
    m = mask_ref[0][..., None]
    out_ref[0] = table_ref[...] * m


def _size_body(pos_ref, mask_ref, size_ref, out_ref):
    del pos_ref
    m = mask_ref[0][..., None]
    s = size_ref[0, 0, :]
    out_ref[0] = s[None, None, :] * m


def kernel(mask, precomputed_encodings):
    B, H, W = mask.shape
    half = precomputed_encodings.shape[-1]

    table2 = precomputed_encodings.reshape(H * W, half)
    size_enc = _make_sc_prelude(B, H, W, half)(mask, table2)

    Hb = 64
    grid = (H // Hb, B)
    out_shape = jax.ShapeDtypeStruct((B, H, W, 2 * half), jnp.float32)
    pos = pl.pallas_call(
        _pos_body,
        grid=grid,
        in_specs=[
            pl.BlockSpec((1, Hb, W), lambda h, b: (b, h, 0)),
            pl.BlockSpec((Hb, W, half), lambda h, b: (h, 0, 0)),
        ],
        out_specs=pl.BlockSpec((1, Hb, W, half), lambda h, b: (b, h, 0, 0)),
        out_shape=out_shape,
    )(mask, precomputed_encodings)
    out = pl.pallas_call(
        _size_body,
        grid=grid,
        in_specs=[
            pl.BlockSpec(memory_space=pl.ANY),
            pl.BlockSpec((1, Hb, W), lambda h, b: (b, h, 0)),
            pl.BlockSpec((1, 1, half), lambda h, b: (b, 0, 0)),
        ],
        out_specs=pl.BlockSpec((1, Hb, W, half), lambda h, b: (b, h, 0, 1)),
        out_shape=out_shape,
        input_output_aliases={0: 0},
    )(pos, mask, size_enc)
    return out


# speedup vs baseline: 1.1751x; 1.1751x over previous
import functools

import jax
import jax.numpy as jnp
from jax import lax
from jax.experimental import pallas as pl
from jax.experimental.pallas import tpu as pltpu
from jax.experimental.pallas import tpu_sc as plsc

_L = 16


def _lane_allmax(v, scratch):
    iota = jnp.arange(_L, dtype=jnp.int32)
    for k in (8, 4, 2, 1):
        scratch[...] = v
        g = plsc.load_gather(scratch, [jnp.bitwise_xor(iota, k)])
        v = jnp.maximum(v, g)
    return v


def _sc_prelude_body(B, H, W, half, mask_hbm, table_hbm, out_hbm,
                     mask_v, rs_v, stage_v, perm_v, idx_v, rows_v,
                     shared_v, sem):
    nc = 2
    c = lax.axis_index("c")
    s = lax.axis_index("s")
    b = c * (B // nc) + lax.rem(s, 8)
    hhalf = s // 8
    rows_per_w = H // 2
    nchunk = W // _L
    iota = jnp.arange(_L, dtype=jnp.int32)

    pltpu.sync_copy(mask_hbm.at[b, pl.ds(hhalf * rows_per_w, rows_per_w)],
                    mask_v)

    def h_body(h, cacc):
        chunks = [mask_v[h, pl.ds(_L * j, _L)] for j in range(nchunk)]
        rtot = chunks[0]
        for ch in chunks[1:]:
            rtot = rtot + ch
        rs_v[h, :] = rtot
        return tuple(a + ch for a, ch in zip(cacc, chunks))

    zero = jnp.zeros((_L,), jnp.float32)
    cacc = lax.fori_loop(0, rows_per_w, h_body, (zero,) * nchunk)

    rmax = zero
    for g in range(rows_per_w // _L):
        rws = g * _L + iota
        tot = plsc.load_gather(rs_v, [rws, jnp.full((_L,), 0, jnp.int32)])
        for j in range(1, _L):
            tot = tot + plsc.load_gather(
                rs_v, [rws, jnp.full((_L,), j, jnp.int32)])
        rmax = jnp.maximum(rmax, tot)

    for j in range(nchunk):
        stage_v[pl.ds(_L * j, _L)] = cacc[j]
    stage_v[pl.ds(_L * nchunk, _L)] = rmax
    pltpu.sync_copy(stage_v, shared_v.at[s])
    plsc.subcore_barrier()

    @pl.when(s < 8)
    def _():
        pltpu.sync_copy(shared_v.at[s + 8], stage_v)
        cmax = cacc[0] + stage_v[pl.ds(0, _L)]
        for j in range(1, nchunk):
            cmax = jnp.maximum(cmax, cacc[j] + stage_v[pl.ds(_L * j, _L)])
        s1v = _lane_allmax(cmax, perm_v)
        rmax2 = jnp.maximum(rmax, stage_v[pl.ds(_L * nchunk, _L)])
        s2v = _lane_allmax(rmax2, perm_v)

        s1i = jnp.clip(s1v.astype(jnp.int32), 0, H - 1)
        s2i = jnp.clip(s2v.astype(jnp.int32), 0, W - 1)
        idx_v[...] = s1i * W + s2i
        pltpu.async_copy(table_hbm.at[idx_v], rows_v, sem).wait()
        pltpu.sync_copy(rows_v.at[0], out_hbm.at[b, 0])


@functools.lru_cache(maxsize=None)
def _make_sc_prelude(B, H, W, half):
    mesh = plsc.VectorSubcoreMesh(core_axis_name="c", subcore_axis_name="s")
    nstage = W // _L + 1
    return functools.partial(
        pl.kernel,
        mesh=mesh,
        compiler_params=pltpu.CompilerParams(needs_layout_passes=False),
        out_type=jax.ShapeDtypeStruct((B, 1, half), jnp.float32),
        scratch_types=[
            pltpu.VMEM((H // 2, W), jnp.float32),
            pltpu.VMEM((H // 2, _L), jnp.float32),
            pltpu.VMEM((nstage * _L,), jnp.float32),
            pltpu.VMEM((_L,), jnp.float32),
            pltpu.VMEM((_L,), jnp.int32),
            pltpu.VMEM((_L, half), jnp.float32),
            pltpu.VMEM_SHARED((16, nstage * _L), jnp.float32),
            pltpu.SemaphoreType.DMA,
        ],
    )(functools.partial(_sc_prelude_body, B, H, W, half))


def _dense_body(mask_ref, table_ref, size_ref, out_ref):
    m = mask_ref[0][..., None]
    t = table_ref[...]
    s = size_ref[0, 0, :]
    half = t.shape[-1]
    out_ref[0, :, :, :half] = t * m
    out_ref[0, :, :, half:] = s[None, None, :] * m


def kernel(mask, precomputed_encodings):
    B, H, W = mask.shape
    half = precomputed_encodings.shape[-1]

    table2 = precomputed_encodings.reshape(H * W, half)
    size_enc = _make_sc_prelude(B, H, W, half)(mask, table2)

    Hb = 64
    grid = (H // Hb, B)
    out = pl.pallas_call(
        _dense_body,
        grid=grid,
        in_specs=[
            pl.BlockSpec((1, Hb, W), lambda h, b: (b, h, 0)),
            pl.BlockSpec((Hb, W, half), lambda h, b: (h, 0, 0)),
            pl.BlockSpec((1, 1, half), lambda h, b: (b, 0, 0)),
        ],
        out_specs=pl.BlockSpec((1, Hb, W, 2 * half), lambda h, b: (b, h, 0, 0)),
        out_shape=jax.ShapeDtypeStruct((B, H, W, 2 * half), jnp.float32),
    )(mask, precomputed_encodings, size_enc)
    return out
